# baseline (speedup 1.0000x reference)
import functools

import jax
import jax.numpy as jnp
from jax import lax
from jax.experimental import pallas as pl
from jax.experimental.pallas import tpu as pltpu
from jax.experimental.pallas import tpu_sc as plsc

_D = 6
_HID = 128
_OUT = 2 * _D * 2 + _D
_TBL_W = 48


def _floordiv6(t):
    return (t * 43691) >> 18


def _table_body(x_ref, scale_ref, w_ref, tbl_ref):
    x2 = x_ref[...]
    x3 = x2.reshape(x2.shape[0] // _D, _D, _HID)
    xm = jnp.mean(x3, axis=1)
    s = jnp.sum(xm, axis=1, keepdims=True)
    ss = jnp.sum(xm * xm, axis=1, keepdims=True)
    u = jnp.dot(xm * scale_ref[...], w_ref[...],
                preferred_element_type=jnp.float32)
    li = lax.broadcasted_iota(jnp.int32, (1, _TBL_W), 1)
    tbl_ref[...] = u + jnp.where(li == 32, s, 0.0) + jnp.where(li == 33, ss, 0.0)


def _build_table(xr, scale_half, w_pad, bn):
    n = xr.shape[0] // _D
    return pl.pallas_call(
        _table_body,
        grid=(n // bn,),
        in_specs=[
            pl.BlockSpec((bn * _D, _HID), lambda i: (i, 0)),
            pl.BlockSpec((1, _HID), lambda i: (0, 0)),
            pl.BlockSpec((_HID, _TBL_W), lambda i: (0, 0)),
        ],
        out_specs=pl.BlockSpec((bn, _TBL_W), lambda i: (i, 0)),
        out_shape=jax.ShapeDtypeStruct((n, _TBL_W), jnp.float32),
        compiler_params=pltpu.CompilerParams(dimension_semantics=("parallel",)),
    )(xr, scale_half, w_pad)


def _consts_body(w_ref, scale_ref, bias_ref, b_ref, sc_ref):
    w = w_ref[...]
    s = jnp.sum(w * scale_ref[...], axis=0, keepdims=True)
    c = jnp.sum(w * bias_ref[...], axis=0, keepdims=True) + b_ref[...]
    sc_ref[...] = jnp.concatenate([s, c], axis=0)


def _build_consts(w_pad_full, scale2d, bias2d, b_pad):
    return pl.pallas_call(
        _consts_body,
        out_shape=jax.ShapeDtypeStruct((2, _TBL_W), jnp.float32),
    )(w_pad_full, scale2d, bias2d, b_pad)


def _sc_gather(tx, te, row2d, col2d, window):
    num_pairs = row2d.shape[1]
    n_x = tx.shape[0]
    n_e = te.shape[0]
    mesh = plsc.VectorSubcoreMesh(core_axis_name="c", subcore_axis_name="s")

    @functools.partial(
        pl.kernel,
        out_type=jax.ShapeDtypeStruct((num_pairs, 128), jnp.float32),
        mesh=mesh,
        scratch_types=[
            pltpu.VMEM_SHARED((n_x, _TBL_W), jnp.float32),
            pltpu.VMEM_SHARED((n_e, _TBL_W), jnp.float32),
            pltpu.SemaphoreType.DMA,
            pltpu.VMEM((window, _TBL_W), jnp.float32),
            pltpu.VMEM((window, _TBL_W), jnp.float32),
        ],
        compiler_params=pltpu.CompilerParams(use_tc_tiling_on_sc=False),
    )
    def gather_kernel(tx_hbm, te_hbm, row_hbm, col_hbm, r_hbm,
                      tx_sp, te_sp, sem, gx_v, ge_v):
        @pl.when(lax.axis_index("s") == 0)
        def _stage():
            pltpu.async_copy(tx_hbm, tx_sp, sem).wait()
            pltpu.async_copy(te_hbm, te_sp, sem).wait()

        plsc.subcore_barrier()

        def body(row_v, col_v, r_v):
            pltpu.sync_copy(tx_sp.at[row_v.at[0]], gx_v)
            pltpu.sync_copy(te_sp.at[col_v.at[0]], ge_v)

            @pl.loop(0, window, step=4)
            def _(i):
                for u in range(4):
                    r_v[i + u, pl.ds(0, 16)] = (gx_v[i + u, pl.ds(0, 16)]
                                                + ge_v[i + u, pl.ds(0, 16)])
                    r_v[i + u, pl.ds(16, 16)] = (gx_v[i + u, pl.ds(16, 16)]
                                                 + ge_v[i + u, pl.ds(16, 16)])
                    r_v[i + u, pl.ds(32, 16)] = (gx_v[i + u, pl.ds(32, 16)]
                                                 + ge_v[i + u, pl.ds(32, 16)])

        pltpu.emit_pipeline(
            body,
            grid=(num_pairs // window,),
            in_specs=[
                pl.BlockSpec((1, window), lambda i: (0, i)),
                pl.BlockSpec((1, window), lambda i: (0, i)),
            ],
            out_specs=[
                pl.BlockSpec((window, 128), lambda i: (i, 0)),
            ],
            core_axis_name=("c", "s"),
            dimension_semantics=(pltpu.PARALLEL,),
        )(row_hbm, col_hbm, r_hbm)

    return gather_kernel(tx, te, row2d, col2d)


def _pairs_body(r_ref, sc_ref, attr_ref):
    r = r_ref[...][:, 0:_TBL_W]
    s_row = sc_ref[0:1, :]
    c_row = sc_ref[1:2, :]
    mu = r[:, 32:33] * (1.0 / 256.0)
    ex2 = r[:, 33:34] * (1.0 / 256.0)
    inv = lax.rsqrt(ex2 - mu * mu + 1e-5)
    z = (r - mu * s_row) * inv + c_row
    p = jax.nn.sigmoid(z)

    f32, bf16 = jnp.float32, jnp.bfloat16
    m = lax.broadcasted_iota(jnp.int32, (_TBL_W, 36), 0)
    t = lax.broadcasted_iota(jnp.int32, (_TBL_W, 36), 1)
    i = _floordiv6(t)
    j = t - 6 * i
    ea0 = (m == 2 * i).astype(bf16)
    eb0 = (m == 12 + 2 * j).astype(bf16)
    ea1 = (m == 2 * i + 1).astype(bf16)
    eb1 = (m == 13 + 2 * j).astype(bf16)
    ec = ((m == 24 + i) & (i == j)).astype(bf16)

    p_bf = p.astype(bf16)

    def mm(bmat):
        return jnp.dot(p_bf, bmat, preferred_element_type=f32)

    hs = mm(ea0) * mm(eb0) + mm(ea1) * mm(eb1) + mm(ec)
    attr_ref[...] = hs


def _build_attrs(rsum, sc, bp):
    num_pairs = rsum.shape[0]
    return pl.pallas_call(
        _pairs_body,
        grid=(num_pairs // bp,),
        in_specs=[
            pl.BlockSpec((bp, 128), lambda i: (i, 0)),
            pl.BlockSpec((2, _TBL_W), lambda i: (0, 0)),
        ],
        out_specs=pl.BlockSpec((bp, 36), lambda i: (i, 0)),
        out_shape=jax.ShapeDtypeStruct((num_pairs, 36), jnp.float32),
        compiler_params=pltpu.CompilerParams(dimension_semantics=("parallel",)),
    )(rsum, sc)


def _idx_body(row_ref, col_ref, out_ref):
    ngrp = out_ref.shape[0] // 144
    i32, f32, bf16 = jnp.int32, jnp.float32, jnp.bfloat16
    r_i = lax.broadcasted_iota(i32, (144, 128), 0)
    l_i = lax.broadcasted_iota(i32, (144, 128), 1)
    c = r_i >> 1
    isev = (r_i & 1) == 0
    f = 128 * c + l_i
    t = f - 36 * ((f * 58255) >> 21)
    td6 = _floordiv6(t)
    tdil = jnp.where(isev, td6, t - 6 * td6).astype(f32)
    r_a = lax.broadcasted_iota(i32, (144, 256), 0)
    k_a = lax.broadcasted_iota(i32, (144, 256), 1)
    c_a = r_a >> 1
    isev_a = (r_a & 1) == 0
    c1 = (36 * k_a) >> 7
    a1 = c_a == c1
    a2 = c_a == c1 + 1
    a1b = a1.astype(bf16)
    a2b = a2.astype(bf16)
    k_b = lax.broadcasted_iota(i32, (256, 128), 0)
    l_b = lax.broadcasted_iota(i32, (256, 128), 1)
    s1 = (36 * k_b) & 127
    b1 = ((l_b >= s1) & (l_b < s1 + 36)).astype(bf16)
    b2 = (l_b < s1 - 92).astype(bf16)

    def mm(a, bmat):
        return jnp.dot(a, bmat, preferred_element_type=f32)

    for g in range(ngrp):
        row = row_ref[g]
        col = col_ref[g]
        w_h = jnp.where(isev_a, row >> 7, col >> 7).astype(bf16)
        w_l = jnp.where(isev_a, row & 127, col & 127).astype(bf16)
        out = (768.0 * (mm(a1b * w_h, b1) + mm(a2b * w_h, b2))
               + 6.0 * (mm(a1b * w_l, b1) + mm(a2b * w_l, b2))
               + tdil)
        out_ref[g * 144:(g + 1) * 144, :] = out.astype(i32)


def _build_index(hyperedge_index, grp):
    num_pairs = hyperedge_index.shape[1]
    ngroups = num_pairs // 256
    row3 = hyperedge_index[0].reshape(ngroups, 1, 256)
    col3 = hyperedge_index[1].reshape(ngroups, 1, 256)
    out2 = pl.pallas_call(
        _idx_body,
        grid=(ngroups // grp,),
        in_specs=[
            pl.BlockSpec((grp, 1, 256), lambda i: (i, 0, 0)),
            pl.BlockSpec((grp, 1, 256), lambda i: (i, 0, 0)),
        ],
        out_specs=pl.BlockSpec((144 * grp, 128), lambda i: (i, 0)),
        out_shape=jax.ShapeDtypeStruct((144 * ngroups, 128), hyperedge_index.dtype),
        compiler_params=pltpu.CompilerParams(dimension_semantics=("parallel",)),
    )(row3, col3)
    return (out2.reshape(num_pairs * 36 // 128, 2, 128)
            .transpose(1, 0, 2)
            .reshape(2, num_pairs * 36))


def _pick_block(n, candidates):
    for c in candidates:
        if n % c == 0:
            return c
    return n


def kernel(x, e, hyperedge_index, ln_scale, ln_bias, W, b):
    num_nodes = x.shape[0] // _D
    num_edges = e.shape[0] // _D
    num_pairs = hyperedge_index.shape[1]

    w_x = jnp.pad(W[:_HID], ((0, 0), (0, _TBL_W - _OUT)))
    w_e = jnp.pad(W[_HID:], ((0, 0), (0, _TBL_W - _OUT)))
    w_full = jnp.pad(W, ((0, 0), (0, _TBL_W - _OUT)))
    b_pad = jnp.pad(b, (0, _TBL_W - _OUT)).reshape(1, _TBL_W)
    scale_x = ln_scale[:_HID].reshape(1, _HID)
    scale_e = ln_scale[_HID:].reshape(1, _HID)
    scale2d = ln_scale.reshape(2 * _HID, 1)
    bias2d = ln_bias.reshape(2 * _HID, 1)

    bn_x = _pick_block(num_nodes, (1000, 500, 250, 200, 100, 50, 25, 10, 5, 2))
    bn_e = _pick_block(num_edges, (1000, 500, 250, 200, 100, 50, 25, 10, 5, 2))
    tx = _build_table(x, scale_x, w_x, bn_x)
    te = _build_table(e, scale_e, w_e, bn_e)
    sc = _build_consts(w_full, scale2d, bias2d, b_pad)

    window = _pick_block(num_pairs, (128, 64, 32, 16, 8))
    row2d = hyperedge_index[0:1]
    col2d = hyperedge_index[1:2]
    rsum = _sc_gather(tx, te, row2d, col2d, window)

    bp = _pick_block(num_pairs, (6400, 3200, 1600, 800, 400, 200, 100, 50, 25, 10, 8))
    attrs = _build_attrs(rsum, sc, bp)

    grp = _pick_block(num_pairs // 256, (10, 5, 25, 2, 1))
    idx = _build_index(hyperedge_index, grp)

    return (idx, attrs.reshape(num_pairs * _D * _D))


# speedup vs baseline: 1.0078x; 1.0078x over previous
import functools

import jax
import jax.numpy as jnp
from jax import lax
from jax.experimental import pallas as pl
from jax.experimental.pallas import tpu as pltpu
from jax.experimental.pallas import tpu_sc as plsc

_D = 6
_HID = 128
_OUT = 2 * _D * 2 + _D
_TBL_W = 48


def _floordiv6(t):
    return (t * 43691) >> 18


def _table_body(x_ref, scale_ref, w_ref, tbl_ref):
    x2 = x_ref[...]
    x3 = x2.reshape(x2.shape[0] // _D, _D, _HID)
    xm = jnp.mean(x3, axis=1)
    s = jnp.sum(xm, axis=1, keepdims=True)
    ss = jnp.sum(xm * xm, axis=1, keepdims=True)
    u = jnp.dot(xm * scale_ref[...], w_ref[...],
                preferred_element_type=jnp.float32)
    li = lax.broadcasted_iota(jnp.int32, (1, _TBL_W), 1)
    tbl_ref[...] = u + jnp.where(li == 32, s, 0.0) + jnp.where(li == 33, ss, 0.0)


def _build_table(xr, scale_half, w_pad, bn):
    n = xr.shape[0] // _D
    return pl.pallas_call(
        _table_body,
        grid=(n // bn,),
        in_specs=[
            pl.BlockSpec((bn * _D, _HID), lambda i: (i, 0)),
            pl.BlockSpec((1, _HID), lambda i: (0, 0)),
            pl.BlockSpec((_HID, _TBL_W), lambda i: (0, 0)),
        ],
        out_specs=pl.BlockSpec((bn, _TBL_W), lambda i: (i, 0)),
        out_shape=jax.ShapeDtypeStruct((n, _TBL_W), jnp.float32),
        compiler_params=pltpu.CompilerParams(dimension_semantics=("parallel",)),
    )(xr, scale_half, w_pad)


def _consts_body(w_ref, scale_ref, bias_ref, b_ref, sc_ref):
    w = w_ref[...]
    s = jnp.sum(w * scale_ref[...], axis=0, keepdims=True)
    c = jnp.sum(w * bias_ref[...], axis=0, keepdims=True) + b_ref[...]
    sc_ref[...] = jnp.concatenate([s, c], axis=0)


def _build_consts(w_pad_full, scale2d, bias2d, b_pad):
    return pl.pallas_call(
        _consts_body,
        out_shape=jax.ShapeDtypeStruct((2, _TBL_W), jnp.float32),
    )(w_pad_full, scale2d, bias2d, b_pad)


def _sc_gather(tx, te, row2d, col2d, window):
    num_pairs = row2d.shape[1]
    n_x = tx.shape[0]
    n_e = te.shape[0]
    mesh = plsc.VectorSubcoreMesh(core_axis_name="c", subcore_axis_name="s")

    @functools.partial(
        pl.kernel,
        out_type=jax.ShapeDtypeStruct((num_pairs, 128), jnp.float32),
        mesh=mesh,
        scratch_types=[
            pltpu.VMEM_SHARED((n_x, _TBL_W), jnp.float32),
            pltpu.VMEM_SHARED((n_e, _TBL_W), jnp.float32),
            pltpu.SemaphoreType.DMA,
            pltpu.VMEM((window, _TBL_W), jnp.float32),
            pltpu.VMEM((window, _TBL_W), jnp.float32),
            pltpu.SemaphoreType.DMA,
            pltpu.SemaphoreType.DMA,
        ],
        compiler_params=pltpu.CompilerParams(use_tc_tiling_on_sc=False),
    )
    def gather_kernel(tx_hbm, te_hbm, row_hbm, col_hbm, r_hbm,
                      tx_sp, te_sp, sem, gx_v, ge_v, gsem1, gsem2):
        @pl.when(lax.axis_index("s") == 0)
        def _stage():
            pltpu.async_copy(tx_hbm, tx_sp, sem).wait()
            pltpu.async_copy(te_hbm, te_sp, sem).wait()

        plsc.subcore_barrier()

        def body(row_v, col_v, r_v):
            cp1 = pltpu.async_copy(tx_sp.at[row_v.at[0]], gx_v, gsem1)
            cp2 = pltpu.async_copy(te_sp.at[col_v.at[0]], ge_v, gsem2)
            cp1.wait()
            cp2.wait()

            @pl.loop(0, window, step=4)
            def _(i):
                for u in range(4):
                    r_v[i + u, pl.ds(0, 16)] = (gx_v[i + u, pl.ds(0, 16)]
                                                + ge_v[i + u, pl.ds(0, 16)])
                    r_v[i + u, pl.ds(16, 16)] = (gx_v[i + u, pl.ds(16, 16)]
                                                 + ge_v[i + u, pl.ds(16, 16)])
                    r_v[i + u, pl.ds(32, 16)] = (gx_v[i + u, pl.ds(32, 16)]
                                                 + ge_v[i + u, pl.ds(32, 16)])

        pltpu.emit_pipeline(
            body,
            grid=(num_pairs // window,),
            in_specs=[
                pl.BlockSpec((1, window), lambda i: (0, i)),
                pl.BlockSpec((1, window), lambda i: (0, i)),
            ],
            out_specs=[
                pl.BlockSpec((window, 128), lambda i: (i, 0)),
            ],
            core_axis_name=("c", "s"),
            dimension_semantics=(pltpu.PARALLEL,),
        )(row_hbm, col_hbm, r_hbm)

    return gather_kernel(tx, te, row2d, col2d)


def _pairs_body(r_ref, sc_ref, attr_ref):
    r = r_ref[...][:, 0:_TBL_W]
    s_row = sc_ref[0:1, :]
    c_row = sc_ref[1:2, :]
    mu = r[:, 32:33] * (1.0 / 256.0)
    ex2 = r[:, 33:34] * (1.0 / 256.0)
    inv = lax.rsqrt(ex2 - mu * mu + 1e-5)
    z = (r - mu * s_row) * inv + c_row
    p = jax.nn.sigmoid(z)

    f32, bf16 = jnp.float32, jnp.bfloat16
    m = lax.broadcasted_iota(jnp.int32, (_TBL_W, 36), 0)
    t = lax.broadcasted_iota(jnp.int32, (_TBL_W, 36), 1)
    i = _floordiv6(t)
    j = t - 6 * i
    ea0 = (m == 2 * i).astype(bf16)
    eb0 = (m == 12 + 2 * j).astype(bf16)
    ea1 = (m == 2 * i + 1).astype(bf16)
    eb1 = (m == 13 + 2 * j).astype(bf16)
    ec = ((m == 24 + i) & (i == j)).astype(bf16)

    p_bf = p.astype(bf16)

    def mm(bmat):
        return jnp.dot(p_bf, bmat, preferred_element_type=f32)

    hs = mm(ea0) * mm(eb0) + mm(ea1) * mm(eb1) + mm(ec)
    attr_ref[...] = hs


def _build_attrs(rsum, sc, bp):
    num_pairs = rsum.shape[0]
    return pl.pallas_call(
        _pairs_body,
        grid=(num_pairs // bp,),
        in_specs=[
            pl.BlockSpec((bp, 128), lambda i: (i, 0)),
            pl.BlockSpec((2, _TBL_W), lambda i: (0, 0)),
        ],
        out_specs=pl.BlockSpec((bp, 36), lambda i: (i, 0)),
        out_shape=jax.ShapeDtypeStruct((num_pairs, 36), jnp.float32),
        compiler_params=pltpu.CompilerParams(dimension_semantics=("parallel",)),
    )(rsum, sc)


def _idx_body(row_ref, col_ref, out_ref):
    ngrp = out_ref.shape[0] // 144
    i32, f32, bf16 = jnp.int32, jnp.float32, jnp.bfloat16
    r_i = lax.broadcasted_iota(i32, (144, 128), 0)
    l_i = lax.broadcasted_iota(i32, (144, 128), 1)
    c = r_i >> 1
    isev = (r_i & 1) == 0
    f = 128 * c + l_i
    t = f - 36 * ((f * 58255) >> 21)
    td6 = _floordiv6(t)
    tdil = jnp.where(isev, td6, t - 6 * td6).astype(f32)
    r_a = lax.broadcasted_iota(i32, (144, 256), 0)
    k_a = lax.broadcasted_iota(i32, (144, 256), 1)
    c_a = r_a >> 1
    isev_a = (r_a & 1) == 0
    c1 = (36 * k_a) >> 7
    a1 = c_a == c1
    a2 = c_a == c1 + 1
    a1b = a1.astype(bf16)
    a2b = a2.astype(bf16)
    k_b = lax.broadcasted_iota(i32, (256, 128), 0)
    l_b = lax.broadcasted_iota(i32, (256, 128), 1)
    s1 = (36 * k_b) & 127
    b1 = ((l_b >= s1) & (l_b < s1 + 36)).astype(bf16)
    b2 = (l_b < s1 - 92).astype(bf16)

    def mm(a, bmat):
        return jnp.dot(a, bmat, preferred_element_type=f32)

    for g in range(ngrp):
        row = row_ref[g]
        col = col_ref[g]
        w_h = jnp.where(isev_a, row >> 7, col >> 7).astype(bf16)
        w_l = jnp.where(isev_a, row & 127, col & 127).astype(bf16)
        out = (768.0 * (mm(a1b * w_h, b1) + mm(a2b * w_h, b2))
               + 6.0 * (mm(a1b * w_l, b1) + mm(a2b * w_l, b2))
               + tdil)
        out_ref[g * 144:(g + 1) * 144, :] = out.astype(i32)


def _build_index(hyperedge_index, grp):
    num_pairs = hyperedge_index.shape[1]
    ngroups = num_pairs // 256
    row3 = hyperedge_index[0].reshape(ngroups, 1, 256)
    col3 = hyperedge_index[1].reshape(ngroups, 1, 256)
    out2 = pl.pallas_call(
        _idx_body,
        grid=(ngroups // grp,),
        in_specs=[
            pl.BlockSpec((grp, 1, 256), lambda i: (i, 0, 0)),
            pl.BlockSpec((grp, 1, 256), lambda i: (i, 0, 0)),
        ],
        out_specs=pl.BlockSpec((144 * grp, 128), lambda i: (i, 0)),
        out_shape=jax.ShapeDtypeStruct((144 * ngroups, 128), hyperedge_index.dtype),
        compiler_params=pltpu.CompilerParams(dimension_semantics=("parallel",)),
    )(row3, col3)
    return (out2.reshape(num_pairs * 36 // 128, 2, 128)
            .transpose(1, 0, 2)
            .reshape(2, num_pairs * 36))


def _pick_block(n, candidates):
    for c in candidates:
        if n % c == 0:
            return c
    return n


def kernel(x, e, hyperedge_index, ln_scale, ln_bias, W, b):
    num_nodes = x.shape[0] // _D
    num_edges = e.shape[0] // _D
    num_pairs = hyperedge_index.shape[1]

    w_x = jnp.pad(W[:_HID], ((0, 0), (0, _TBL_W - _OUT)))
    w_e = jnp.pad(W[_HID:], ((0, 0), (0, _TBL_W - _OUT)))
    w_full = jnp.pad(W, ((0, 0), (0, _TBL_W - _OUT)))
    b_pad = jnp.pad(b, (0, _TBL_W - _OUT)).reshape(1, _TBL_W)
    scale_x = ln_scale[:_HID].reshape(1, _HID)
    scale_e = ln_scale[_HID:].reshape(1, _HID)
    scale2d = ln_scale.reshape(2 * _HID, 1)
    bias2d = ln_bias.reshape(2 * _HID, 1)

    bn_x = _pick_block(num_nodes, (1000, 500, 250, 200, 100, 50, 25, 10, 5, 2))
    bn_e = _pick_block(num_edges, (1000, 500, 250, 200, 100, 50, 25, 10, 5, 2))
    tx = _build_table(x, scale_x, w_x, bn_x)
    te = _build_table(e, scale_e, w_e, bn_e)
    sc = _build_consts(w_full, scale2d, bias2d, b_pad)

    window = _pick_block(num_pairs, (128, 64, 32, 16, 8))
    row2d = hyperedge_index[0:1]
    col2d = hyperedge_index[1:2]
    rsum = _sc_gather(tx, te, row2d, col2d, window)

    bp = _pick_block(num_pairs, (6400, 3200, 1600, 800, 400, 200, 100, 50, 25, 10, 8))
    attrs = _build_attrs(rsum, sc, bp)

    grp = _pick_block(num_pairs // 256, (10, 5, 25, 2, 1))
    idx = _build_index(hyperedge_index, grp)

    return (idx, attrs.reshape(num_pairs * _D * _D))
